# Initial kernel scaffold; baseline (speedup 1.0000x reference)
import jax
import jax.numpy as jnp
from jax.experimental import pallas as pl


def kernel(fm2, fm3, fm4, fm5, boxes1, boxes2):
    raise NotImplementedError("write your pallas kernel here")


# speedup vs baseline: 5.3704x; 5.3704x over previous
import jax
import jax.numpy as jnp
from jax import lax
from jax.experimental import pallas as pl
from jax.experimental.pallas import tpu as pltpu
from jax.experimental.pallas import tpu_sc as plsc

OUT = 7
C = 256
M = 1024
NC, NS, L = 2, 16, 16
NW = NC * NS
BOX_PER_W = M // NW
NPOS = OUT * OUT
SLOTS = 56
TAB = 4 * SLOTS
OUT_WORDS = C * NPOS

_GRID = tuple((i + 0.5) / OUT for i in range(OUT))


def _sc_body(table, boxes_t, out_flat, coords, idx_all, w_all,
             rows0, rows1, rows2, rows3, out_v, sem):
    wid = lax.axis_index("s") * NC + lax.axis_index("c")
    box0 = wid * BOX_PER_W
    iota = lax.iota(jnp.int32, L)
    zeros_i = jnp.zeros((L,), jnp.int32)
    ones_i = jnp.full((L,), 1, jnp.int32)

    @pl.loop(0, 2)
    def _groups(g):
        gb = box0 + g * L
        for c4 in range(4):
            pltpu.sync_copy(boxes_t.at[c4, pl.ds(gb, L)], coords.at[c4])
        x1 = coords[0]
        y1 = coords[1]
        x2 = coords[2]
        y2 = coords[3]
        area = (x2 - x1) * (y2 - y1)
        lvm2 = (jnp.where(area >= 12544.0, ones_i, zeros_i)
                + jnp.where(area >= 50176.0, ones_i, zeros_i)
                + jnp.where(area >= 200704.0, ones_i, zeros_i))
        stridef = jnp.left_shift(jnp.full((L,), 4, jnp.int32),
                                 lvm2).astype(jnp.float32)
        wi = jnp.right_shift(jnp.full((L,), 128, jnp.int32), lvm2)
        hw = wi * wi
        base_rows = jnp.where(
            lvm2 == 0, zeros_i,
            jnp.where(lvm2 == 1, jnp.full((L,), 32768, jnp.int32),
                      jnp.where(lvm2 == 2, jnp.full((L,), 40960, jnp.int32),
                                jnp.full((L,), 43008, jnp.int32))))
        bvec = jnp.full((L,), gb, jnp.int32)
        rowbase = base_rows + jnp.where(bvec >= 512, hw, zeros_i)
        wim1 = wi - ones_i

        x1s = x1 / stridef
        x2s = x2 / stridef
        y1s = y1 / stridef
        y2s = y2 / stridef
        cols0, cols1, wxl, omwxl = [], [], [], []
        rb0, rb1, wyl, omwyl = [], [], [], []
        for o in range(OUT):
            t = _GRID[o]
            px = x1s + t * (x2s - x1s)
            x0t = px.astype(jnp.int32)
            wx = px - x0t.astype(jnp.float32)
            cols0.append(jnp.minimum(x0t, wim1))
            cols1.append(jnp.minimum(x0t + 1, wim1))
            wxl.append(wx)
            omwxl.append(1.0 - wx)
            py = y1s + t * (y2s - y1s)
            y0t = py.astype(jnp.int32)
            wy = py - y0t.astype(jnp.float32)
            rb0.append(rowbase + jnp.minimum(y0t, wim1) * wi)
            rb1.append(rowbase + jnp.minimum(y0t + 1, wim1) * wi)
            wyl.append(wy)
            omwyl.append(1.0 - wy)

        tb = (g * L + iota) * TAB
        p = 0
        for oy in range(OUT):
            for ox in range(OUT):
                vals = (
                    (rb0[oy] + cols0[ox], omwyl[oy] * omwxl[ox]),
                    (rb0[oy] + cols1[ox], omwyl[oy] * wxl[ox]),
                    (rb1[oy] + cols0[ox], wyl[oy] * omwxl[ox]),
                    (rb1[oy] + cols1[ox], wyl[oy] * wxl[ox]),
                )
                for c4, (iv, wv) in enumerate(vals):
                    fidx = tb + (c4 * SLOTS + p)
                    plsc.store_scatter(idx_all, [fidx], iv)
                    plsc.store_scatter(w_all, [fidx], wv)
                p += 1
        for p in range(NPOS, SLOTS):
            for c4 in range(4):
                plsc.store_scatter(idx_all, [tb + (c4 * SLOTS + p)], zeros_i)

    cbase = [(iota + k * L) * NPOS for k in range(C // L)]
    rows_refs = (rows0, rows1, rows2, rows3)

    @pl.loop(0, BOX_PER_W)
    def _boxes(b):
        descs = [
            pltpu.async_copy(
                table.at[idx_all.at[pl.ds(b * TAB + c4 * SLOTS, SLOTS)]],
                rows_refs[c4], sem)
            for c4 in range(4)
        ]
        for d in descs:
            d.wait()
        wbase = jnp.full((L,), b * TAB, jnp.int32)

        @pl.loop(0, NPOS)
        def _pos(p):
            wp = wbase + p
            w00 = plsc.load_gather(w_all, [wp])
            w01 = plsc.load_gather(w_all, [wp + SLOTS])
            w10 = plsc.load_gather(w_all, [wp + 2 * SLOTS])
            w11 = plsc.load_gather(w_all, [wp + 3 * SLOTS])
            for k in range(C // L):
                sl = pl.ds(k * L, L)
                acc = (rows0[p, sl] * w00 + rows1[p, sl] * w01
                       + rows2[p, sl] * w10 + rows3[p, sl] * w11)
                plsc.store_scatter(out_v, [cbase[k] + p], acc)

        pltpu.sync_copy(
            out_v, out_flat.at[pl.ds((box0 + b) * OUT_WORDS, OUT_WORDS)])


_mesh = plsc.VectorSubcoreMesh(
    core_axis_name="c", subcore_axis_name="s", num_cores=NC, num_subcores=NS)

_run = pl.kernel(
    _sc_body,
    out_type=jax.ShapeDtypeStruct((M * OUT_WORDS,), jnp.float32),
    mesh=_mesh,
    compiler_params=pltpu.CompilerParams(needs_layout_passes=False),
    scratch_types=[
        pltpu.VMEM((4, L), jnp.float32),
        pltpu.VMEM((BOX_PER_W * TAB,), jnp.int32),
        pltpu.VMEM((BOX_PER_W * TAB,), jnp.float32),
        pltpu.VMEM((SLOTS, C), jnp.float32),
        pltpu.VMEM((SLOTS, C), jnp.float32),
        pltpu.VMEM((SLOTS, C), jnp.float32),
        pltpu.VMEM((SLOTS, C), jnp.float32),
        pltpu.VMEM((OUT_WORDS,), jnp.float32),
        pltpu.SemaphoreType.DMA,
    ],
)


@jax.jit
def kernel(fm2, fm3, fm4, fm5, boxes1, boxes2):
    tabs = [jnp.transpose(fm, (0, 2, 3, 1)).reshape(-1, C)
            for fm in (fm2, fm3, fm4, fm5)]
    table = jnp.concatenate(tabs, axis=0)
    boxes_t = jnp.concatenate([boxes1, boxes2], axis=0).T
    out_flat = _run(table, boxes_t)
    return out_flat.reshape(M, C, OUT, OUT)
